# baseline (speedup 1.0000x reference)
import functools

import jax
import jax.numpy as jnp
from jax import lax
from jax.experimental import pallas as pl

N_TOKENS = 32768
D_IN = 1024
N_EXPERTS = 8
BT = 1024


def _router_body(z_ref, w_ref, b_ref, out_ref):
    z = z_ref[...]
    w = w_ref[...]
    logits = lax.dot_general(z, w, (((1,), (1,)), ((), ())),
                             preferred_element_type=jnp.float32)
    logits = logits + b_ref[...]
    m = jnp.max(logits, axis=-1, keepdims=True)
    e = jnp.exp(logits - m)
    s = jnp.sum(e, axis=-1, keepdims=True)
    out_ref[...] = e / s


def kernel(z, W, b):
    n_tokens = z.shape[0]
    grid = (n_tokens // BT,)
    b2 = b.reshape(1, N_EXPERTS)
    return pl.pallas_call(
        _router_body,
        grid=grid,
        in_specs=[
            pl.BlockSpec((BT, D_IN), lambda i: (i, 0)),
            pl.BlockSpec((N_EXPERTS, D_IN), lambda i: (0, 0)),
            pl.BlockSpec((1, N_EXPERTS), lambda i: (0, 0)),
        ],
        out_specs=pl.BlockSpec((BT, N_EXPERTS), lambda i: (i, 0)),
        out_shape=jax.ShapeDtypeStruct((n_tokens, N_EXPERTS), jnp.float32),
    )(z, W, b2)


# speedup vs baseline: 1.1546x; 1.1546x over previous
import functools

import jax
import jax.numpy as jnp
from jax import lax
from jax.experimental import pallas as pl

N_TOKENS = 32768
D_IN = 1024
N_EXPERTS = 8
BT = 4096


def _router_body(z_ref, w_ref, b_ref, out_ref):
    z = z_ref[...]
    w = w_ref[...]
    logits = lax.dot_general(z, w, (((1,), (1,)), ((), ())),
                             preferred_element_type=jnp.float32)
    logits = logits + b_ref[...]
    m = jnp.max(logits, axis=-1, keepdims=True)
    e = jnp.exp(logits - m)
    s = jnp.sum(e, axis=-1, keepdims=True)
    out_ref[...] = e / s


def kernel(z, W, b):
    n_tokens = z.shape[0]
    grid = (n_tokens // BT,)
    b2 = b.reshape(1, N_EXPERTS)
    return pl.pallas_call(
        _router_body,
        grid=grid,
        in_specs=[
            pl.BlockSpec((BT, D_IN), lambda i: (i, 0)),
            pl.BlockSpec((N_EXPERTS, D_IN), lambda i: (0, 0)),
            pl.BlockSpec((1, N_EXPERTS), lambda i: (0, 0)),
        ],
        out_specs=pl.BlockSpec((BT, N_EXPERTS), lambda i: (i, 0)),
        out_shape=jax.ShapeDtypeStruct((n_tokens, N_EXPERTS), jnp.float32),
    )(z, W, b2)
